# Initial kernel scaffold; baseline (speedup 1.0000x reference)
import jax
import jax.numpy as jnp
from jax.experimental import pallas as pl


def kernel(x, edge_index, edge_type, batch, W1, root1, b1, W2, root2, b2):
    raise NotImplementedError("write your pallas kernel here")


# speedup vs baseline: 37.1218x; 37.1218x over previous
import functools

import jax
import jax.numpy as jnp
from jax import lax
from jax.experimental import pallas as pl
from jax.experimental.pallas import tpu as pltpu
from jax.experimental.pallas import tpu_sc as plsc

N = 10000
E = 320000
IN = 128
H = 128
R = 8
G = 16
NR = N * R
NR_PAD = 81920

_INFO = plsc.get_sparse_core_info()
NC = _INFO.num_cores
NS = _INFO.num_subcores
NW = NC * NS

E_PER_TILE = E // NW
E_PER_SUB = E // NS
CHUNK = 80
N_CHUNKS = E_PER_TILE // CHUNK
N_CHUNKS_CNT = E_PER_SUB // CHUNK
N_PAD = 10240
ROWS_PER_TILE = N_PAD // NS
ZNR_PER_TILE = NR_PAD // NS

_MESH = plsc.VectorSubcoreMesh(core_axis_name="c", subcore_axis_name="s")
_SC_PARAMS = pltpu.CompilerParams(needs_layout_passes=False)


@functools.partial(
    pl.kernel,
    out_type=jax.ShapeDtypeStruct((E,), jnp.float32),
    mesh=_MESH,
    scratch_types=[
        pltpu.VMEM((E_PER_SUB,), jnp.int32),
        pltpu.VMEM((CHUNK,), jnp.int32),
        pltpu.VMEM((CHUNK,), jnp.float32),
        pltpu.VMEM((NR_PAD,), jnp.float32),
        pltpu.VMEM((E_PER_TILE,), jnp.float32),
        pltpu.VMEM_SHARED((NR_PAD,), jnp.float32),
    ],
    compiler_params=_SC_PARAMS,
)
def _sc_weights(cidx_hbm, zeros_hbm, w_hbm, cbuf, ibuf, ones, cpriv, wbuf,
                scnt):
    cid = lax.axis_index("c")
    sid = lax.axis_index("s")
    wid = cid * NS + sid

    pltpu.sync_copy(zeros_hbm.at[pl.ds(sid * ZNR_PER_TILE, ZNR_PER_TILE)],
                    scnt.at[pl.ds(sid * ZNR_PER_TILE, ZNR_PER_TILE)])
    for j in range(CHUNK // 16):
        ones[pl.ds(j * 16, 16)] = jnp.full((16,), 1.0, jnp.float32)
    pltpu.sync_copy(cidx_hbm.at[pl.ds(sid * E_PER_SUB, E_PER_SUB)], cbuf)
    plsc.subcore_barrier()

    def count_body(c, _):
        for j in range(CHUNK // 16):
            ibuf[pl.ds(j * 16, 16)] = cbuf[pl.ds(c * CHUNK + j * 16, 16)]
        pltpu.sync_copy(ones, scnt.at[ibuf], add=True)
        return _

    lax.fori_loop(0, N_CHUNKS_CNT, count_body, 0)
    plsc.subcore_barrier()

    pltpu.sync_copy(scnt, cpriv)
    pltpu.sync_copy(cidx_hbm.at[pl.ds(wid * E_PER_TILE, E_PER_TILE)],
                    cbuf.at[pl.ds(0, E_PER_TILE)])

    def w_body(g, _):
        idx16 = cbuf[pl.ds(g * 16, 16)]
        c16 = plsc.load_gather(cpriv, [idx16])
        wbuf[pl.ds(g * 16, 16)] = 1.0 / jnp.maximum(c16, 1.0)
        return _

    lax.fori_loop(0, E_PER_TILE // 16, w_body, 0)
    pltpu.sync_copy(wbuf, w_hbm.at[pl.ds(wid * E_PER_TILE, E_PER_TILE)])


@functools.partial(
    pl.kernel,
    out_type=jax.ShapeDtypeStruct((NC, N_PAD, H), jnp.float32),
    mesh=_MESH,
    scratch_types=[
        pltpu.VMEM((E_PER_TILE,), jnp.int32),
        pltpu.VMEM((E_PER_TILE,), jnp.int32),
        pltpu.VMEM((E_PER_TILE,), jnp.float32),
        pltpu.VMEM((CHUNK,), jnp.int32),
        pltpu.VMEM((CHUNK,), jnp.int32),
        pltpu.VMEM((CHUNK, H), jnp.float32),
        pltpu.SemaphoreType.DMA,
        pltpu.VMEM_SHARED((N_PAD, H), jnp.float32),
    ],
    compiler_params=_SC_PARAMS,
)
def _sc_aggregate(table_hbm, gidx_hbm, dst_hbm, w_hbm, zeros_hbm, out_hbm,
                  gall, dall, wall, gbuf, dbuf, rows, sem, acc):
    cid = lax.axis_index("c")
    sid = lax.axis_index("s")
    wid = cid * NS + sid
    base = wid * E_PER_TILE

    pltpu.sync_copy(zeros_hbm.at[pl.ds(sid * ROWS_PER_TILE, ROWS_PER_TILE)],
                    acc.at[pl.ds(sid * ROWS_PER_TILE, ROWS_PER_TILE)])
    pltpu.sync_copy(gidx_hbm.at[pl.ds(base, E_PER_TILE)], gall)
    pltpu.sync_copy(dst_hbm.at[pl.ds(base, E_PER_TILE)], dall)
    pltpu.sync_copy(w_hbm.at[pl.ds(base, E_PER_TILE)], wall)
    plsc.subcore_barrier()

    def chunk_body(c, _):
        for j in range(CHUNK // 16):
            gbuf[pl.ds(j * 16, 16)] = gall[pl.ds(c * CHUNK + j * 16, 16)]
            dbuf[pl.ds(j * 16, 16)] = dall[pl.ds(c * CHUNK + j * 16, 16)]
        pltpu.async_copy(table_hbm.at[gbuf], rows, sem).wait()

        def scale_body(g, _2):
            w16 = wall[pl.ds(c * CHUNK + g * 16, 16)]
            for j in range(16):
                ws = w16[j]
                k = g * 16 + j
                for col in range(H // 16):
                    rows[k, pl.ds(col * 16, 16)] = (
                        rows[k, pl.ds(col * 16, 16)] * ws)
            return _2

        lax.fori_loop(0, CHUNK // 16, scale_body, 0)
        pltpu.sync_copy(rows, acc.at[dbuf], add=True)
        return _

    lax.fori_loop(0, N_CHUNKS, chunk_body, 0)
    plsc.subcore_barrier()

    pltpu.sync_copy(acc.at[pl.ds(sid * ROWS_PER_TILE, ROWS_PER_TILE)],
                    out_hbm.at[cid, pl.ds(sid * ROWS_PER_TILE, ROWS_PER_TILE)])


BN = 1000


def _mm1_body(x_ref, w_ref, xr_ref, self_ref):
    y = jnp.dot(x_ref[...], w_ref[...], preferred_element_type=jnp.float32)
    xr_ref[...] = y[:, :R * H]
    self_ref[...] = y[:, R * H:]


def _tc_transform(x, w_full):
    return pl.pallas_call(
        _mm1_body,
        grid=(N // BN,),
        in_specs=[
            pl.BlockSpec((BN, IN), lambda i: (i, 0)),
            pl.BlockSpec((IN, R * H + H), lambda i: (0, 0)),
        ],
        out_specs=[
            pl.BlockSpec((BN, R * H), lambda i: (i, 0)),
            pl.BlockSpec((BN, H), lambda i: (i, 0)),
        ],
        out_shape=[
            jax.ShapeDtypeStruct((N, R * H), jnp.float32),
            jax.ShapeDtypeStruct((N, H), jnp.float32),
        ],
    )(x, w_full)


def _mm2_body(self_ref, agg_ref, b_ref, w_ref, xr_ref, self2_ref):
    h = jnp.maximum(
        self_ref[...] + agg_ref[0] + agg_ref[1] + b_ref[...], 0.0)
    y = jnp.dot(h, w_ref[...], preferred_element_type=jnp.float32)
    xr_ref[...] = y[:, :R * H]
    self2_ref[...] = y[:, R * H:]


def _tc_relu_transform(self1, agg, b, w_full):
    return pl.pallas_call(
        _mm2_body,
        grid=(N // BN,),
        in_specs=[
            pl.BlockSpec((BN, H), lambda i: (i, 0)),
            pl.BlockSpec((NC, BN, H), lambda i: (0, i, 0)),
            pl.BlockSpec((1, H), lambda i: (0, 0)),
            pl.BlockSpec((H, R * H + H), lambda i: (0, 0)),
        ],
        out_specs=[
            pl.BlockSpec((BN, R * H), lambda i: (i, 0)),
            pl.BlockSpec((BN, H), lambda i: (i, 0)),
        ],
        out_shape=[
            jax.ShapeDtypeStruct((N, R * H), jnp.float32),
            jax.ShapeDtypeStruct((N, H), jnp.float32),
        ],
    )(self1, agg, b, w_full)


def _final_body(self_ref, agg_ref, b_ref, batch_ref, h_ref, pool_ref):
    i = pl.program_id(0)
    h = jnp.maximum(
        self_ref[...] + agg_ref[0] + agg_ref[1] + b_ref[...], 0.0)
    h_ref[...] = h

    @pl.when(i == 0)
    def _():
        pool_ref[...] = jnp.full((G, H), -jnp.inf, jnp.float32)

    bids = batch_ref[...]
    for g in range(G):
        mg = jnp.max(jnp.where(bids == g, h, -jnp.inf), axis=0,
                     keepdims=True)
        pool_ref[pl.ds(g, 1), :] = jnp.maximum(pool_ref[pl.ds(g, 1), :], mg)


def _tc_final(self2, agg, b, batch2d):
    return pl.pallas_call(
        _final_body,
        grid=(N // BN,),
        in_specs=[
            pl.BlockSpec((BN, H), lambda i: (i, 0)),
            pl.BlockSpec((NC, BN, H), lambda i: (0, i, 0)),
            pl.BlockSpec((1, H), lambda i: (0, 0)),
            pl.BlockSpec((BN, 1), lambda i: (i, 0)),
        ],
        out_specs=[
            pl.BlockSpec((BN, H), lambda i: (i, 0)),
            pl.BlockSpec((G, H), lambda i: (0, 0)),
        ],
        out_shape=[
            jax.ShapeDtypeStruct((N, H), jnp.float32),
            jax.ShapeDtypeStruct((G, H), jnp.float32),
        ],
    )(self2, agg, b, batch2d)


def kernel(x, edge_index, edge_type, batch, W1, root1, b1, W2, root2, b2):
    src = edge_index[0]
    dst = edge_index[1]
    gidx = src * R + edge_type
    cidx = dst * R + edge_type

    zeros_nr = jnp.zeros((NR_PAD,), jnp.float32)
    zeros_nh = jnp.zeros((N_PAD, H), jnp.float32)

    w_edge = _sc_weights(cidx, zeros_nr)

    wfull1 = jnp.concatenate(
        [W1.transpose(1, 0, 2).reshape(IN, R * H), root1], axis=1)
    xr1, self1 = _tc_transform(x, wfull1)
    agg1 = _sc_aggregate(xr1.reshape(NR, H), gidx, dst, w_edge, zeros_nh)

    wfull2 = jnp.concatenate(
        [W2.transpose(1, 0, 2).reshape(H, R * H), root2], axis=1)
    xr2, self2 = _tc_relu_transform(self1, agg1, b1.reshape(1, H), wfull2)
    agg2 = _sc_aggregate(xr2.reshape(NR, H), gidx, dst, w_edge, zeros_nh)

    h, pooled = _tc_final(self2, agg2, b2.reshape(1, H),
                          batch.reshape(N, 1))
    return (h, pooled)


# speedup vs baseline: 44.5399x; 1.1998x over previous
import functools

import jax
import jax.numpy as jnp
from jax import lax
from jax.experimental import pallas as pl
from jax.experimental.pallas import tpu as pltpu
from jax.experimental.pallas import tpu_sc as plsc

N = 10000
E = 320000
IN = 128
H = 128
R = 8
G = 16
NR = N * R
NR_PAD = 81920

_INFO = plsc.get_sparse_core_info()
NC = _INFO.num_cores
NS = _INFO.num_subcores
NW = NC * NS

E_PER_TILE = E // NW
E_PER_SUB = E // NS
CHUNK = 80
N_CHUNKS = E_PER_TILE // CHUNK
N_CHUNKS_CNT = E_PER_SUB // CHUNK
N_PAD = 10240
ROWS_PER_TILE = N_PAD // NS
ZNR_PER_TILE = NR_PAD // NS

_MESH = plsc.VectorSubcoreMesh(core_axis_name="c", subcore_axis_name="s")
_SC_PARAMS = pltpu.CompilerParams(needs_layout_passes=False)


@functools.partial(
    pl.kernel,
    out_type=jax.ShapeDtypeStruct((E,), jnp.float32),
    mesh=_MESH,
    scratch_types=[
        pltpu.VMEM((E_PER_SUB,), jnp.int32),
        pltpu.VMEM((CHUNK,), jnp.int32),
        pltpu.VMEM((CHUNK,), jnp.float32),
        pltpu.VMEM((NR_PAD,), jnp.float32),
        pltpu.VMEM((E_PER_TILE,), jnp.float32),
        pltpu.VMEM_SHARED((NR_PAD,), jnp.float32),
    ],
    compiler_params=_SC_PARAMS,
)
def _sc_weights(cidx_hbm, zeros_hbm, w_hbm, cbuf, ibuf, ones, cpriv, wbuf,
                scnt):
    cid = lax.axis_index("c")
    sid = lax.axis_index("s")
    wid = cid * NS + sid

    pltpu.sync_copy(zeros_hbm.at[pl.ds(sid * ZNR_PER_TILE, ZNR_PER_TILE)],
                    scnt.at[pl.ds(sid * ZNR_PER_TILE, ZNR_PER_TILE)])
    for j in range(CHUNK // 16):
        ones[pl.ds(j * 16, 16)] = jnp.full((16,), 1.0, jnp.float32)
    pltpu.sync_copy(cidx_hbm.at[pl.ds(sid * E_PER_SUB, E_PER_SUB)], cbuf)
    plsc.subcore_barrier()

    def count_body(c, _):
        for j in range(CHUNK // 16):
            ibuf[pl.ds(j * 16, 16)] = cbuf[pl.ds(c * CHUNK + j * 16, 16)]
        pltpu.sync_copy(ones, scnt.at[ibuf], add=True)
        return _

    lax.fori_loop(0, N_CHUNKS_CNT, count_body, 0)
    plsc.subcore_barrier()

    pltpu.sync_copy(scnt, cpriv)
    pltpu.sync_copy(cidx_hbm.at[pl.ds(wid * E_PER_TILE, E_PER_TILE)],
                    cbuf.at[pl.ds(0, E_PER_TILE)])

    def w_body(g, _):
        idx16 = cbuf[pl.ds(g * 16, 16)]
        c16 = plsc.load_gather(cpriv, [idx16])
        wbuf[pl.ds(g * 16, 16)] = 1.0 / jnp.maximum(c16, 1.0)
        return _

    lax.fori_loop(0, E_PER_TILE // 16, w_body, 0)
    pltpu.sync_copy(wbuf, w_hbm.at[pl.ds(wid * E_PER_TILE, E_PER_TILE)])


@functools.partial(
    pl.kernel,
    out_type=jax.ShapeDtypeStruct((NC, N_PAD, H), jnp.float32),
    mesh=_MESH,
    scratch_types=[
        pltpu.VMEM((3 * CHUNK,), jnp.int32),
        pltpu.VMEM((CHUNK,), jnp.int32),
        pltpu.VMEM((CHUNK,), jnp.int32),
        pltpu.VMEM((CHUNK, H), jnp.float32),
        pltpu.VMEM((3 * CHUNK,), jnp.int32),
        pltpu.VMEM((CHUNK,), jnp.int32),
        pltpu.VMEM((CHUNK,), jnp.int32),
        pltpu.VMEM((CHUNK, H), jnp.float32),
        pltpu.SemaphoreType.DMA,
        pltpu.SemaphoreType.DMA,
        pltpu.SemaphoreType.DMA,
        pltpu.SemaphoreType.DMA,
        pltpu.VMEM_SHARED((N_PAD, H), jnp.float32),
    ],
    compiler_params=_SC_PARAMS,
)
def _sc_aggregate(table_hbm, edata_hbm, zeros_hbm, out_hbm,
                  ebuf_a, gbuf_a, dbuf_a, rows_a, ebuf_b, gbuf_b, dbuf_b,
                  rows_b, sg_a, ss_a, sg_b, ss_b, acc):
    cid = lax.axis_index("c")
    sid = lax.axis_index("s")
    wid = cid * NS + sid

    pltpu.sync_copy(zeros_hbm.at[pl.ds(sid * ROWS_PER_TILE, ROWS_PER_TILE)],
                    acc.at[pl.ds(sid * ROWS_PER_TILE, ROWS_PER_TILE)])
    plsc.subcore_barrier()

    def stage_and_gather(c, ebuf, gbuf, dbuf, rows, sg):
        off = pl.multiple_of((wid * N_CHUNKS + c) * (3 * CHUNK), 8)
        pltpu.sync_copy(edata_hbm.at[pl.ds(off, 3 * CHUNK)], ebuf)
        for j in range(CHUNK // 16):
            gbuf[pl.ds(j * 16, 16)] = ebuf[pl.ds(j * 16, 16)]
            dbuf[pl.ds(j * 16, 16)] = ebuf[pl.ds(CHUNK + j * 16, 16)]
        pltpu.async_copy(table_hbm.at[gbuf], rows, sg)

    def scale(ebuf, rows):
        def scale_body(g, _2):
            w16 = plsc.bitcast(ebuf[pl.ds(2 * CHUNK + g * 16, 16)],
                               jnp.float32)
            for j in range(16):
                ws = w16[j]
                k = g * 16 + j
                for col in range(H // 16):
                    rows[k, pl.ds(col * 16, 16)] = (
                        rows[k, pl.ds(col * 16, 16)] * ws)
            return _2

        lax.fori_loop(0, CHUNK // 16, scale_body, 0)

    def process(ebuf, gbuf, dbuf, rows, sg, ss):
        pltpu.make_async_copy(table_hbm.at[gbuf], rows, sg).wait()
        scale(ebuf, rows)
        pltpu.async_copy(rows, acc.at[dbuf], ss, add=True)

    def scatter_wait(dbuf, rows, ss):
        pltpu.make_async_copy(rows, acc.at[dbuf], ss).wait()

    stage_and_gather(0, ebuf_a, gbuf_a, dbuf_a, rows_a, sg_a)
    stage_and_gather(1, ebuf_b, gbuf_b, dbuf_b, rows_b, sg_b)

    def pair_body(k, carry):
        process(ebuf_a, gbuf_a, dbuf_a, rows_a, sg_a, ss_a)
        process(ebuf_b, gbuf_b, dbuf_b, rows_b, sg_b, ss_b)
        scatter_wait(dbuf_a, rows_a, ss_a)
        stage_and_gather(2 * k + 2, ebuf_a, gbuf_a, dbuf_a, rows_a, sg_a)

        @pl.when(k < N_CHUNKS // 2 - 1)
        def _prefetch_b():
            scatter_wait(dbuf_b, rows_b, ss_b)
            stage_and_gather(2 * k + 3, ebuf_b, gbuf_b, dbuf_b, rows_b, sg_b)

        return carry

    lax.fori_loop(0, N_CHUNKS // 2, pair_body, 0)
    process(ebuf_a, gbuf_a, dbuf_a, rows_a, sg_a, ss_a)
    scatter_wait(dbuf_a, rows_a, ss_a)
    scatter_wait(dbuf_b, rows_b, ss_b)
    plsc.subcore_barrier()

    pltpu.sync_copy(acc.at[pl.ds(sid * ROWS_PER_TILE, ROWS_PER_TILE)],
                    out_hbm.at[cid, pl.ds(sid * ROWS_PER_TILE, ROWS_PER_TILE)])


BN = 1000


def _mm1_body(x_ref, w_ref, xr_ref, self_ref):
    y = jnp.dot(x_ref[...], w_ref[...], preferred_element_type=jnp.float32)
    xr_ref[...] = y[:, :R * H]
    self_ref[...] = y[:, R * H:]


def _tc_transform(x, w_full):
    return pl.pallas_call(
        _mm1_body,
        grid=(N // BN,),
        in_specs=[
            pl.BlockSpec((BN, IN), lambda i: (i, 0)),
            pl.BlockSpec((IN, R * H + H), lambda i: (0, 0)),
        ],
        out_specs=[
            pl.BlockSpec((BN, R * H), lambda i: (i, 0)),
            pl.BlockSpec((BN, H), lambda i: (i, 0)),
        ],
        out_shape=[
            jax.ShapeDtypeStruct((N, R * H), jnp.float32),
            jax.ShapeDtypeStruct((N, H), jnp.float32),
        ],
    )(x, w_full)


def _mm2_body(self_ref, agg_ref, b_ref, w_ref, xr_ref, self2_ref):
    h = jnp.maximum(
        self_ref[...] + agg_ref[0] + agg_ref[1] + b_ref[...], 0.0)
    y = jnp.dot(h, w_ref[...], preferred_element_type=jnp.float32)
    xr_ref[...] = y[:, :R * H]
    self2_ref[...] = y[:, R * H:]


def _tc_relu_transform(self1, agg, b, w_full):
    return pl.pallas_call(
        _mm2_body,
        grid=(N // BN,),
        in_specs=[
            pl.BlockSpec((BN, H), lambda i: (i, 0)),
            pl.BlockSpec((NC, BN, H), lambda i: (0, i, 0)),
            pl.BlockSpec((1, H), lambda i: (0, 0)),
            pl.BlockSpec((H, R * H + H), lambda i: (0, 0)),
        ],
        out_specs=[
            pl.BlockSpec((BN, R * H), lambda i: (i, 0)),
            pl.BlockSpec((BN, H), lambda i: (i, 0)),
        ],
        out_shape=[
            jax.ShapeDtypeStruct((N, R * H), jnp.float32),
            jax.ShapeDtypeStruct((N, H), jnp.float32),
        ],
    )(self1, agg, b, w_full)


def _final_body(self_ref, agg_ref, b_ref, batch_ref, h_ref, pool_ref):
    i = pl.program_id(0)
    h = jnp.maximum(
        self_ref[...] + agg_ref[0] + agg_ref[1] + b_ref[...], 0.0)
    h_ref[...] = h

    @pl.when(i == 0)
    def _():
        pool_ref[...] = jnp.full((G, H), -jnp.inf, jnp.float32)

    bids = batch_ref[...]
    for g in range(G):
        mg = jnp.max(jnp.where(bids == g, h, -jnp.inf), axis=0,
                     keepdims=True)
        pool_ref[pl.ds(g, 1), :] = jnp.maximum(pool_ref[pl.ds(g, 1), :], mg)


def _tc_final(self2, agg, b, batch2d):
    return pl.pallas_call(
        _final_body,
        grid=(N // BN,),
        in_specs=[
            pl.BlockSpec((BN, H), lambda i: (i, 0)),
            pl.BlockSpec((NC, BN, H), lambda i: (0, i, 0)),
            pl.BlockSpec((1, H), lambda i: (0, 0)),
            pl.BlockSpec((BN, 1), lambda i: (i, 0)),
        ],
        out_specs=[
            pl.BlockSpec((BN, H), lambda i: (i, 0)),
            pl.BlockSpec((G, H), lambda i: (0, 0)),
        ],
        out_shape=[
            jax.ShapeDtypeStruct((N, H), jnp.float32),
            jax.ShapeDtypeStruct((G, H), jnp.float32),
        ],
    )(self2, agg, b, batch2d)


def kernel(x, edge_index, edge_type, batch, W1, root1, b1, W2, root2, b2):
    src = edge_index[0]
    dst = edge_index[1]
    gidx = src * R + edge_type
    cidx = dst * R + edge_type

    zeros_nr = jnp.zeros((NR_PAD,), jnp.float32)
    zeros_nh = jnp.zeros((N_PAD, H), jnp.float32)

    w_edge = _sc_weights(cidx, zeros_nr)
    w_bits = lax.bitcast_convert_type(w_edge, jnp.int32)
    edata = (jnp.stack([gidx, dst, w_bits])
             .reshape(3, NW * N_CHUNKS, CHUNK)
             .transpose(1, 0, 2)
             .reshape(-1))

    wfull1 = jnp.concatenate(
        [W1.transpose(1, 0, 2).reshape(IN, R * H), root1], axis=1)
    xr1, self1 = _tc_transform(x, wfull1)
    agg1 = _sc_aggregate(xr1.reshape(NR, H), edata, zeros_nh)

    wfull2 = jnp.concatenate(
        [W2.transpose(1, 0, 2).reshape(H, R * H), root2], axis=1)
    xr2, self2 = _tc_relu_transform(self1, agg1, b1.reshape(1, H), wfull2)
    agg2 = _sc_aggregate(xr2.reshape(NR, H), edata, zeros_nh)

    h, pooled = _tc_final(self2, agg2, b2.reshape(1, H),
                          batch.reshape(N, 1))
    return (h, pooled)


# speedup vs baseline: 50.1636x; 1.1263x over previous
import functools

import jax
import jax.numpy as jnp
from jax import lax
from jax.experimental import pallas as pl
from jax.experimental.pallas import tpu as pltpu
from jax.experimental.pallas import tpu_sc as plsc

N = 10000
E = 320000
IN = 128
H = 128
R = 8
G = 16
NR = N * R
NR_PAD = 81920

_INFO = plsc.get_sparse_core_info()
NC = _INFO.num_cores
NS = _INFO.num_subcores
NW = NC * NS

E_PER_TILE = E // NW
E_PER_SUB = E // NS
CHUNK = 80
N_CHUNKS = E_PER_TILE // CHUNK
N_CHUNKS_CNT = E_PER_SUB // CHUNK
N_PAD = 10240
ROWS_PER_TILE = N_PAD // NS
ZNR_PER_TILE = NR_PAD // NS

_MESH = plsc.VectorSubcoreMesh(core_axis_name="c", subcore_axis_name="s")
_SC_PARAMS = pltpu.CompilerParams(needs_layout_passes=False)


@functools.partial(
    pl.kernel,
    out_type=jax.ShapeDtypeStruct((E,), jnp.float32),
    mesh=_MESH,
    scratch_types=[
        pltpu.VMEM((E_PER_SUB,), jnp.int32),
        pltpu.VMEM((CHUNK,), jnp.int32),
        pltpu.VMEM((CHUNK,), jnp.float32),
        pltpu.VMEM((NR_PAD,), jnp.float32),
        pltpu.VMEM((E_PER_TILE,), jnp.float32),
        pltpu.VMEM_SHARED((NR_PAD,), jnp.float32),
    ],
    compiler_params=_SC_PARAMS,
)
def _sc_weights(cidx_hbm, zeros_hbm, w_hbm, cbuf, ibuf, ones, cpriv, wbuf,
                scnt):
    cid = lax.axis_index("c")
    sid = lax.axis_index("s")
    wid = cid * NS + sid

    pltpu.sync_copy(zeros_hbm.at[pl.ds(sid * ZNR_PER_TILE, ZNR_PER_TILE)],
                    scnt.at[pl.ds(sid * ZNR_PER_TILE, ZNR_PER_TILE)])
    for j in range(CHUNK // 16):
        ones[pl.ds(j * 16, 16)] = jnp.full((16,), 1.0, jnp.float32)
    pltpu.sync_copy(cidx_hbm.at[pl.ds(sid * E_PER_SUB, E_PER_SUB)], cbuf)
    plsc.subcore_barrier()

    def count_body(c, _):
        for j in range(CHUNK // 16):
            ibuf[pl.ds(j * 16, 16)] = cbuf[pl.ds(c * CHUNK + j * 16, 16)]
        pltpu.sync_copy(ones, scnt.at[ibuf], add=True)
        return _

    lax.fori_loop(0, N_CHUNKS_CNT, count_body, 0)
    plsc.subcore_barrier()

    pltpu.sync_copy(scnt, cpriv)
    pltpu.sync_copy(cidx_hbm.at[pl.ds(wid * E_PER_TILE, E_PER_TILE)],
                    cbuf.at[pl.ds(0, E_PER_TILE)])

    def w_body(g, _):
        idx16 = cbuf[pl.ds(g * 16, 16)]
        c16 = plsc.load_gather(cpriv, [idx16])
        wbuf[pl.ds(g * 16, 16)] = 1.0 / jnp.maximum(c16, 1.0)
        return _

    lax.fori_loop(0, E_PER_TILE // 16, w_body, 0)
    pltpu.sync_copy(wbuf, w_hbm.at[pl.ds(wid * E_PER_TILE, E_PER_TILE)])


@functools.partial(
    pl.kernel,
    out_type=jax.ShapeDtypeStruct((NC, N_PAD, H), jnp.float32),
    mesh=_MESH,
    scratch_types=[
        pltpu.VMEM((3 * CHUNK,), jnp.int32),
        pltpu.VMEM((CHUNK,), jnp.int32),
        pltpu.VMEM((CHUNK,), jnp.int32),
        pltpu.VMEM((CHUNK, H), jnp.float32),
        pltpu.VMEM((3 * CHUNK,), jnp.int32),
        pltpu.VMEM((CHUNK,), jnp.int32),
        pltpu.VMEM((CHUNK,), jnp.int32),
        pltpu.VMEM((CHUNK, H), jnp.float32),
        pltpu.SemaphoreType.DMA,
        pltpu.SemaphoreType.DMA,
        pltpu.SemaphoreType.DMA,
        pltpu.SemaphoreType.DMA,
        pltpu.VMEM_SHARED((N_PAD, H), jnp.float32),
    ],
    compiler_params=_SC_PARAMS,
)
def _sc_aggregate(table_hbm, edata_hbm, zeros_hbm, out_hbm,
                  ebuf_a, gbuf_a, dbuf_a, rows_a, ebuf_b, gbuf_b, dbuf_b,
                  rows_b, sg_a, ss_a, sg_b, ss_b, acc):
    cid = lax.axis_index("c")
    sid = lax.axis_index("s")
    wid = cid * NS + sid

    pltpu.sync_copy(zeros_hbm.at[pl.ds(sid * ROWS_PER_TILE, ROWS_PER_TILE)],
                    acc.at[pl.ds(sid * ROWS_PER_TILE, ROWS_PER_TILE)])
    plsc.subcore_barrier()

    def stage_and_gather(c, ebuf, gbuf, dbuf, rows, sg):
        off = pl.multiple_of((wid * N_CHUNKS + c) * (3 * CHUNK), 8)
        pltpu.sync_copy(edata_hbm.at[pl.ds(off, 3 * CHUNK)], ebuf)
        for j in range(CHUNK // 16):
            gbuf[pl.ds(j * 16, 16)] = ebuf[pl.ds(j * 16, 16)]
            dbuf[pl.ds(j * 16, 16)] = ebuf[pl.ds(CHUNK + j * 16, 16)]
        pltpu.async_copy(table_hbm.at[gbuf], rows, sg)

    def scale(ebuf, rows):
        def scale_body(g, _2):
            w16 = plsc.bitcast(ebuf[pl.ds(2 * CHUNK + g * 16, 16)],
                               jnp.float32)
            for j in range(16):
                ws = w16[j]
                k = g * 16 + j
                for col in range(H // 16):
                    rows[k, pl.ds(col * 16, 16)] = (
                        rows[k, pl.ds(col * 16, 16)] * ws)
            return _2

        lax.fori_loop(0, CHUNK // 16, scale_body, 0)

    def process(ebuf, gbuf, dbuf, rows, sg, ss):
        pltpu.make_async_copy(table_hbm.at[gbuf], rows, sg).wait()
        scale(ebuf, rows)
        pltpu.async_copy(rows, acc.at[dbuf], ss, add=True)

    def scatter_wait(dbuf, rows, ss):
        pltpu.make_async_copy(rows, acc.at[dbuf], ss).wait()

    stage_and_gather(0, ebuf_a, gbuf_a, dbuf_a, rows_a, sg_a)
    stage_and_gather(1, ebuf_b, gbuf_b, dbuf_b, rows_b, sg_b)

    def pair_body(k, carry):
        process(ebuf_a, gbuf_a, dbuf_a, rows_a, sg_a, ss_a)
        process(ebuf_b, gbuf_b, dbuf_b, rows_b, sg_b, ss_b)
        scatter_wait(dbuf_a, rows_a, ss_a)
        stage_and_gather(2 * k + 2, ebuf_a, gbuf_a, dbuf_a, rows_a, sg_a)

        @pl.when(k < N_CHUNKS // 2 - 1)
        def _prefetch_b():
            scatter_wait(dbuf_b, rows_b, ss_b)
            stage_and_gather(2 * k + 3, ebuf_b, gbuf_b, dbuf_b, rows_b, sg_b)

        return carry

    lax.fori_loop(0, N_CHUNKS // 2, pair_body, 0)
    process(ebuf_a, gbuf_a, dbuf_a, rows_a, sg_a, ss_a)
    scatter_wait(dbuf_a, rows_a, ss_a)
    scatter_wait(dbuf_b, rows_b, ss_b)
    plsc.subcore_barrier()

    pltpu.sync_copy(acc.at[pl.ds(sid * ROWS_PER_TILE, ROWS_PER_TILE)],
                    out_hbm.at[cid, pl.ds(sid * ROWS_PER_TILE, ROWS_PER_TILE)])


BN = 1000


def _mm1_body(x_ref, w_ref, xr_ref, self_ref):
    y = jnp.dot(x_ref[...], w_ref[...], preferred_element_type=jnp.float32)
    for r in range(R):
        xr_ref[r] = y[:, r * H:(r + 1) * H]
    self_ref[...] = y[:, R * H:]


def _tc_transform(x, w_full):
    return pl.pallas_call(
        _mm1_body,
        grid=(N // BN,),
        in_specs=[
            pl.BlockSpec((BN, IN), lambda i: (i, 0)),
            pl.BlockSpec((IN, R * H + H), lambda i: (0, 0)),
        ],
        out_specs=[
            pl.BlockSpec((R, BN, H), lambda i: (0, i, 0)),
            pl.BlockSpec((BN, H), lambda i: (i, 0)),
        ],
        out_shape=[
            jax.ShapeDtypeStruct((R, N, H), jnp.float32),
            jax.ShapeDtypeStruct((N, H), jnp.float32),
        ],
    )(x, w_full)


def _mm2_body(self_ref, agg_ref, b_ref, w_ref, xr_ref, self2_ref):
    h = jnp.maximum(
        self_ref[...] + agg_ref[0] + agg_ref[1] + b_ref[...], 0.0)
    y = jnp.dot(h, w_ref[...], preferred_element_type=jnp.float32)
    for r in range(R):
        xr_ref[r] = y[:, r * H:(r + 1) * H]
    self2_ref[...] = y[:, R * H:]


def _tc_relu_transform(self1, agg, b, w_full):
    return pl.pallas_call(
        _mm2_body,
        grid=(N // BN,),
        in_specs=[
            pl.BlockSpec((BN, H), lambda i: (i, 0)),
            pl.BlockSpec((NC, BN, H), lambda i: (0, i, 0)),
            pl.BlockSpec((1, H), lambda i: (0, 0)),
            pl.BlockSpec((H, R * H + H), lambda i: (0, 0)),
        ],
        out_specs=[
            pl.BlockSpec((R, BN, H), lambda i: (0, i, 0)),
            pl.BlockSpec((BN, H), lambda i: (i, 0)),
        ],
        out_shape=[
            jax.ShapeDtypeStruct((R, N, H), jnp.float32),
            jax.ShapeDtypeStruct((N, H), jnp.float32),
        ],
    )(self1, agg, b, w_full)


def _final_body(self_ref, agg_ref, b_ref, batch_ref, h_ref, pool_ref):
    i = pl.program_id(0)
    h = jnp.maximum(
        self_ref[...] + agg_ref[0] + agg_ref[1] + b_ref[...], 0.0)
    h_ref[...] = h

    @pl.when(i == 0)
    def _():
        pool_ref[...] = jnp.full((G, H), -jnp.inf, jnp.float32)

    bids = batch_ref[...]
    for g in range(G):
        mg = jnp.max(jnp.where(bids == g, h, -jnp.inf), axis=0,
                     keepdims=True)
        pool_ref[pl.ds(g, 1), :] = jnp.maximum(pool_ref[pl.ds(g, 1), :], mg)


def _tc_final(self2, agg, b, batch2d):
    return pl.pallas_call(
        _final_body,
        grid=(N // BN,),
        in_specs=[
            pl.BlockSpec((BN, H), lambda i: (i, 0)),
            pl.BlockSpec((NC, BN, H), lambda i: (0, i, 0)),
            pl.BlockSpec((1, H), lambda i: (0, 0)),
            pl.BlockSpec((BN, 1), lambda i: (i, 0)),
        ],
        out_specs=[
            pl.BlockSpec((BN, H), lambda i: (i, 0)),
            pl.BlockSpec((G, H), lambda i: (0, 0)),
        ],
        out_shape=[
            jax.ShapeDtypeStruct((N, H), jnp.float32),
            jax.ShapeDtypeStruct((G, H), jnp.float32),
        ],
    )(self2, agg, b, batch2d)


def kernel(x, edge_index, edge_type, batch, W1, root1, b1, W2, root2, b2):
    src = edge_index[0]
    dst = edge_index[1]
    gidx = edge_type * N + src
    cidx = dst * R + edge_type

    zeros_nr = jnp.zeros((NR_PAD,), jnp.float32)
    zeros_nh = jnp.zeros((N_PAD, H), jnp.float32)

    w_edge = _sc_weights(cidx, zeros_nr)
    w_bits = lax.bitcast_convert_type(w_edge, jnp.int32)
    edata = (jnp.stack([gidx, dst, w_bits])
             .reshape(3, NW * N_CHUNKS, CHUNK)
             .transpose(1, 0, 2)
             .reshape(-1))

    wfull1 = jnp.concatenate(
        [W1.transpose(1, 0, 2).reshape(IN, R * H), root1], axis=1)
    xr1, self1 = _tc_transform(x, wfull1)
    agg1 = _sc_aggregate(xr1.reshape(NR, H), edata, zeros_nh)

    wfull2 = jnp.concatenate(
        [W2.transpose(1, 0, 2).reshape(H, R * H), root2], axis=1)
    xr2, self2 = _tc_relu_transform(self1, agg1, b1.reshape(1, H), wfull2)
    agg2 = _sc_aggregate(xr2.reshape(NR, H), edata, zeros_nh)

    h, pooled = _tc_final(self2, agg2, b2.reshape(1, H),
                          batch.reshape(N, 1))
    return (h, pooled)


# speedup vs baseline: 56.8290x; 1.1329x over previous
import functools

import jax
import jax.numpy as jnp
from jax import lax
from jax.experimental import pallas as pl
from jax.experimental.pallas import tpu as pltpu
from jax.experimental.pallas import tpu_sc as plsc

N = 10000
E = 320000
IN = 128
H = 128
R = 8
G = 16
NR = N * R
NR_PAD = 81920

_INFO = plsc.get_sparse_core_info()
NC = _INFO.num_cores
NS = _INFO.num_subcores
NW = NC * NS

E_PER_TILE = E // NW
E_PER_SUB = E // NS
CHUNK = 80
N_CHUNKS = E_PER_TILE // CHUNK
N_CHUNKS_CNT = E_PER_SUB // CHUNK
N_PAD = 10240
ROWS_PER_TILE = N_PAD // NS
ZNR_PER_TILE = NR_PAD // NS

_MESH = plsc.VectorSubcoreMesh(core_axis_name="c", subcore_axis_name="s")
_SC_PARAMS = pltpu.CompilerParams(needs_layout_passes=False)


@functools.partial(
    pl.kernel,
    out_type=jax.ShapeDtypeStruct((E,), jnp.float32),
    mesh=_MESH,
    scratch_types=[
        pltpu.VMEM((E_PER_SUB,), jnp.int32),
        pltpu.VMEM((CHUNK,), jnp.int32),
        pltpu.VMEM((CHUNK,), jnp.float32),
        pltpu.VMEM((NR_PAD,), jnp.float32),
        pltpu.VMEM((E_PER_TILE,), jnp.float32),
        pltpu.VMEM_SHARED((NR_PAD,), jnp.float32),
    ],
    compiler_params=_SC_PARAMS,
)
def _sc_weights(cidx_hbm, zeros_hbm, w_hbm, cbuf, ibuf, ones, cpriv, wbuf,
                scnt):
    cid = lax.axis_index("c")
    sid = lax.axis_index("s")
    wid = cid * NS + sid

    pltpu.sync_copy(zeros_hbm.at[pl.ds(sid * ZNR_PER_TILE, ZNR_PER_TILE)],
                    scnt.at[pl.ds(sid * ZNR_PER_TILE, ZNR_PER_TILE)])
    for j in range(CHUNK // 16):
        ones[pl.ds(j * 16, 16)] = jnp.full((16,), 1.0, jnp.float32)
    pltpu.sync_copy(cidx_hbm.at[pl.ds(sid * E_PER_SUB, E_PER_SUB)], cbuf)
    plsc.subcore_barrier()

    def count_body(c, _):
        for j in range(CHUNK // 16):
            ibuf[pl.ds(j * 16, 16)] = cbuf[pl.ds(c * CHUNK + j * 16, 16)]
        pltpu.sync_copy(ones, scnt.at[ibuf], add=True)
        return _

    lax.fori_loop(0, N_CHUNKS_CNT, count_body, 0)
    plsc.subcore_barrier()

    pltpu.sync_copy(scnt, cpriv)
    pltpu.sync_copy(cidx_hbm.at[pl.ds(wid * E_PER_TILE, E_PER_TILE)],
                    cbuf.at[pl.ds(0, E_PER_TILE)])

    def w_body(g, _):
        idx16 = cbuf[pl.ds(g * 16, 16)]
        c16 = plsc.load_gather(cpriv, [idx16])
        wbuf[pl.ds(g * 16, 16)] = 1.0 / jnp.maximum(c16, 1.0)
        return _

    lax.fori_loop(0, E_PER_TILE // 16, w_body, 0)
    pltpu.sync_copy(wbuf, w_hbm.at[pl.ds(wid * E_PER_TILE, E_PER_TILE)])


@functools.partial(
    pl.kernel,
    out_type=jax.ShapeDtypeStruct((NC, N_PAD, H), jnp.float32),
    mesh=_MESH,
    scratch_types=(
        [pltpu.VMEM((3 * CHUNK,), jnp.int32)] * 4 +
        [pltpu.VMEM((CHUNK,), jnp.int32)] * 4 +
        [pltpu.VMEM((CHUNK,), jnp.int32)] * 4 +
        [pltpu.VMEM((CHUNK, H), jnp.float32)] * 4 +
        [pltpu.SemaphoreType.DMA] * 4 +
        [pltpu.SemaphoreType.DMA] * 4 +
        [pltpu.VMEM_SHARED((N_PAD, H), jnp.float32)]
    ),
    compiler_params=_SC_PARAMS,
)
def _sc_aggregate(table_hbm, edata_hbm, zeros_hbm, out_hbm,
                  e0, e1, e2, e3, g0, g1, g2, g3, d0, d1, d2, d3,
                  r0, r1, r2, r3, sg0, sg1, sg2, sg3, ss0, ss1, ss2, ss3,
                  acc):
    cid = lax.axis_index("c")
    sid = lax.axis_index("s")
    wid = cid * NS + sid

    pltpu.sync_copy(zeros_hbm.at[pl.ds(sid * ROWS_PER_TILE, ROWS_PER_TILE)],
                    acc.at[pl.ds(sid * ROWS_PER_TILE, ROWS_PER_TILE)])
    plsc.subcore_barrier()

    def stage_and_gather(c, ebuf, gbuf, dbuf, rows, sg):
        off = pl.multiple_of((wid * N_CHUNKS + c) * (3 * CHUNK), 8)
        pltpu.sync_copy(edata_hbm.at[pl.ds(off, 3 * CHUNK)], ebuf)
        for j in range(CHUNK // 16):
            gbuf[pl.ds(j * 16, 16)] = ebuf[pl.ds(j * 16, 16)]
            dbuf[pl.ds(j * 16, 16)] = ebuf[pl.ds(CHUNK + j * 16, 16)]
        pltpu.async_copy(table_hbm.at[gbuf], rows, sg)

    def scale(ebuf, rows):
        def scale_body(g, _2):
            w16 = plsc.bitcast(ebuf[pl.ds(2 * CHUNK + g * 16, 16)],
                               jnp.float32)
            for j in range(16):
                ws = w16[j]
                k = g * 16 + j
                for col in range(H // 16):
                    rows[k, pl.ds(col * 16, 16)] = (
                        rows[k, pl.ds(col * 16, 16)] * ws)
            return _2

        lax.fori_loop(0, CHUNK // 16, scale_body, 0)

    def process(ebuf, gbuf, dbuf, rows, sg, ss):
        pltpu.make_async_copy(table_hbm.at[gbuf], rows, sg).wait()
        scale(ebuf, rows)
        pltpu.async_copy(rows, acc.at[dbuf], ss, add=True)

    def scatter_wait(dbuf, rows, ss):
        pltpu.make_async_copy(rows, acc.at[dbuf], ss).wait()

    bufs = [(e0, g0, d0, r0, sg0, ss0), (e1, g1, d1, r1, sg1, ss1),
            (e2, g2, d2, r2, sg2, ss2), (e3, g3, d3, r3, sg3, ss3)]
    DEPTH = 4
    n_full = N_CHUNKS // DEPTH

    for b in range(DEPTH):
        eb, gb, db, rb, sgb, _ = bufs[b]
        stage_and_gather(b, eb, gb, db, rb, sgb)

    def rot_body(k, carry):
        for b in range(DEPTH):
            eb, gb, db, rb, sgb, ssb = bufs[b]
            process(eb, gb, db, rb, sgb, ssb)
        eb, gb, db, rb, sgb, ssb = bufs[0]
        scatter_wait(db, rb, ssb)
        stage_and_gather(DEPTH * k + DEPTH, eb, gb, db, rb, sgb)
        for b in range(1, DEPTH):
            eb, gb, db, rb, sgb, ssb = bufs[b]

            @pl.when(k < n_full - 1)
            def _prefetch(eb=eb, gb=gb, db=db, rb=rb, sgb=sgb, ssb=ssb,
                          c=DEPTH * k + DEPTH + b):
                scatter_wait(db, rb, ssb)
                stage_and_gather(c, eb, gb, db, rb, sgb)

        return carry

    lax.fori_loop(0, n_full, rot_body, 0)
    e_l, g_l, d_l, r_l, sg_l, ss_l = bufs[0]
    process(e_l, g_l, d_l, r_l, sg_l, ss_l)
    for b in range(DEPTH):
        _, _, db, rb, _, ssb = bufs[b]
        scatter_wait(db, rb, ssb)
    plsc.subcore_barrier()

    pltpu.sync_copy(acc.at[pl.ds(sid * ROWS_PER_TILE, ROWS_PER_TILE)],
                    out_hbm.at[cid, pl.ds(sid * ROWS_PER_TILE, ROWS_PER_TILE)])


BN = 1000


def _mm1_body(x_ref, w_ref, xr_ref, self_ref):
    y = jnp.dot(x_ref[...], w_ref[...], preferred_element_type=jnp.float32)
    for r in range(R):
        xr_ref[r] = y[:, r * H:(r + 1) * H]
    self_ref[...] = y[:, R * H:]


def _tc_transform(x, w_full):
    return pl.pallas_call(
        _mm1_body,
        grid=(N // BN,),
        in_specs=[
            pl.BlockSpec((BN, IN), lambda i: (i, 0)),
            pl.BlockSpec((IN, R * H + H), lambda i: (0, 0)),
        ],
        out_specs=[
            pl.BlockSpec((R, BN, H), lambda i: (0, i, 0)),
            pl.BlockSpec((BN, H), lambda i: (i, 0)),
        ],
        out_shape=[
            jax.ShapeDtypeStruct((R, N, H), jnp.float32),
            jax.ShapeDtypeStruct((N, H), jnp.float32),
        ],
    )(x, w_full)


def _mm2_body(self_ref, agg_ref, b_ref, w_ref, xr_ref, self2_ref):
    h = jnp.maximum(
        self_ref[...] + agg_ref[0] + agg_ref[1] + b_ref[...], 0.0)
    y = jnp.dot(h, w_ref[...], preferred_element_type=jnp.float32)
    for r in range(R):
        xr_ref[r] = y[:, r * H:(r + 1) * H]
    self2_ref[...] = y[:, R * H:]


def _tc_relu_transform(self1, agg, b, w_full):
    return pl.pallas_call(
        _mm2_body,
        grid=(N // BN,),
        in_specs=[
            pl.BlockSpec((BN, H), lambda i: (i, 0)),
            pl.BlockSpec((NC, BN, H), lambda i: (0, i, 0)),
            pl.BlockSpec((1, H), lambda i: (0, 0)),
            pl.BlockSpec((H, R * H + H), lambda i: (0, 0)),
        ],
        out_specs=[
            pl.BlockSpec((R, BN, H), lambda i: (0, i, 0)),
            pl.BlockSpec((BN, H), lambda i: (i, 0)),
        ],
        out_shape=[
            jax.ShapeDtypeStruct((R, N, H), jnp.float32),
            jax.ShapeDtypeStruct((N, H), jnp.float32),
        ],
    )(self1, agg, b, w_full)


def _final_body(self_ref, agg_ref, b_ref, batch_ref, h_ref, pool_ref):
    i = pl.program_id(0)
    h = jnp.maximum(
        self_ref[...] + agg_ref[0] + agg_ref[1] + b_ref[...], 0.0)
    h_ref[...] = h

    @pl.when(i == 0)
    def _():
        pool_ref[...] = jnp.full((G, H), -jnp.inf, jnp.float32)

    bids = batch_ref[...]
    for g in range(G):
        mg = jnp.max(jnp.where(bids == g, h, -jnp.inf), axis=0,
                     keepdims=True)
        pool_ref[pl.ds(g, 1), :] = jnp.maximum(pool_ref[pl.ds(g, 1), :], mg)


def _tc_final(self2, agg, b, batch2d):
    return pl.pallas_call(
        _final_body,
        grid=(N // BN,),
        in_specs=[
            pl.BlockSpec((BN, H), lambda i: (i, 0)),
            pl.BlockSpec((NC, BN, H), lambda i: (0, i, 0)),
            pl.BlockSpec((1, H), lambda i: (0, 0)),
            pl.BlockSpec((BN, 1), lambda i: (i, 0)),
        ],
        out_specs=[
            pl.BlockSpec((BN, H), lambda i: (i, 0)),
            pl.BlockSpec((G, H), lambda i: (0, 0)),
        ],
        out_shape=[
            jax.ShapeDtypeStruct((N, H), jnp.float32),
            jax.ShapeDtypeStruct((G, H), jnp.float32),
        ],
    )(self2, agg, b, batch2d)


def kernel(x, edge_index, edge_type, batch, W1, root1, b1, W2, root2, b2):
    src = edge_index[0]
    dst = edge_index[1]
    gidx = edge_type * N + src
    cidx = dst * R + edge_type

    zeros_nr = jnp.zeros((NR_PAD,), jnp.float32)
    zeros_nh = jnp.zeros((N_PAD, H), jnp.float32)

    w_edge = _sc_weights(cidx, zeros_nr)
    w_bits = lax.bitcast_convert_type(w_edge, jnp.int32)
    edata = (jnp.stack([gidx, dst, w_bits])
             .reshape(3, NW * N_CHUNKS, CHUNK)
             .transpose(1, 0, 2)
             .reshape(-1))

    wfull1 = jnp.concatenate(
        [W1.transpose(1, 0, 2).reshape(IN, R * H), root1], axis=1)
    xr1, self1 = _tc_transform(x, wfull1)
    agg1 = _sc_aggregate(xr1.reshape(NR, H), edata, zeros_nh)

    wfull2 = jnp.concatenate(
        [W2.transpose(1, 0, 2).reshape(H, R * H), root2], axis=1)
    xr2, self2 = _tc_relu_transform(self1, agg1, b1.reshape(1, H), wfull2)
    agg2 = _sc_aggregate(xr2.reshape(NR, H), edata, zeros_nh)

    h, pooled = _tc_final(self2, agg2, b2.reshape(1, H),
                          batch.reshape(N, 1))
    return (h, pooled)


# speedup vs baseline: 60.6184x; 1.0667x over previous
import functools

import jax
import jax.numpy as jnp
from jax import lax
from jax.experimental import pallas as pl
from jax.experimental.pallas import tpu as pltpu
from jax.experimental.pallas import tpu_sc as plsc

N = 10000
E = 320000
IN = 128
H = 128
R = 8
G = 16
NR = N * R
NR_PAD = 81920

_INFO = plsc.get_sparse_core_info()
NC = _INFO.num_cores
NS = _INFO.num_subcores
NW = NC * NS

E_PER_TILE = E // NW
E_PER_SUB = E // NS
CHUNK = 80
N_CHUNKS = E_PER_TILE // CHUNK
N_CHUNKS_CNT = E_PER_SUB // CHUNK
N_PAD = 10240
ROWS_PER_TILE = N_PAD // NS
ZNR_PER_TILE = NR_PAD // NS

_MESH = plsc.VectorSubcoreMesh(core_axis_name="c", subcore_axis_name="s")
_SC_PARAMS = pltpu.CompilerParams(needs_layout_passes=False)


@functools.partial(
    pl.kernel,
    out_type=jax.ShapeDtypeStruct((E,), jnp.float32),
    mesh=_MESH,
    scratch_types=[
        pltpu.VMEM((E_PER_SUB,), jnp.int32),
        pltpu.VMEM((CHUNK,), jnp.int32),
        pltpu.VMEM((CHUNK,), jnp.float32),
        pltpu.VMEM((NR_PAD,), jnp.float32),
        pltpu.VMEM((E_PER_TILE,), jnp.float32),
        pltpu.VMEM_SHARED((NR_PAD,), jnp.float32),
    ],
    compiler_params=_SC_PARAMS,
)
def _sc_weights(cidx_hbm, zeros_hbm, w_hbm, cbuf, ibuf, ones, cpriv, wbuf,
                scnt):
    cid = lax.axis_index("c")
    sid = lax.axis_index("s")
    wid = cid * NS + sid

    pltpu.sync_copy(zeros_hbm.at[pl.ds(sid * ZNR_PER_TILE, ZNR_PER_TILE)],
                    scnt.at[pl.ds(sid * ZNR_PER_TILE, ZNR_PER_TILE)])
    for j in range(CHUNK // 16):
        ones[pl.ds(j * 16, 16)] = jnp.full((16,), 1.0, jnp.float32)
    pltpu.sync_copy(cidx_hbm.at[pl.ds(sid * E_PER_SUB, E_PER_SUB)], cbuf)
    plsc.subcore_barrier()

    def count_body(c, _):
        for j in range(CHUNK // 16):
            ibuf[pl.ds(j * 16, 16)] = cbuf[pl.ds(c * CHUNK + j * 16, 16)]
        pltpu.sync_copy(ones, scnt.at[ibuf], add=True)
        return _

    lax.fori_loop(0, N_CHUNKS_CNT, count_body, 0)
    plsc.subcore_barrier()

    pltpu.sync_copy(scnt, cpriv)
    pltpu.sync_copy(cidx_hbm.at[pl.ds(wid * E_PER_TILE, E_PER_TILE)],
                    cbuf.at[pl.ds(0, E_PER_TILE)])

    def w_body(g, _):
        idx16 = cbuf[pl.ds(g * 16, 16)]
        c16 = plsc.load_gather(cpriv, [idx16])
        wbuf[pl.ds(g * 16, 16)] = 1.0 / jnp.maximum(c16, 1.0)
        return _

    lax.fori_loop(0, E_PER_TILE // 16, w_body, 0)
    pltpu.sync_copy(wbuf, w_hbm.at[pl.ds(wid * E_PER_TILE, E_PER_TILE)])


@functools.partial(
    pl.kernel,
    out_type=jax.ShapeDtypeStruct((NC, N_PAD, H), jnp.float32),
    mesh=_MESH,
    scratch_types=(
        [pltpu.VMEM((3 * CHUNK,), jnp.int32)] * 4 +
        [pltpu.VMEM((CHUNK,), jnp.int32)] * 4 +
        [pltpu.VMEM((CHUNK,), jnp.int32)] * 4 +
        [pltpu.VMEM((CHUNK, H), jnp.float32)] * 4 +
        [pltpu.SemaphoreType.DMA] * 4 +
        [pltpu.SemaphoreType.DMA] * 4 +
        [pltpu.SemaphoreType.DMA] * 4 +
        [pltpu.VMEM_SHARED((N_PAD, H), jnp.float32)]
    ),
    compiler_params=_SC_PARAMS,
)
def _sc_aggregate(table_hbm, edata_hbm, zeros_hbm, out_hbm,
                  e0, e1, e2, e3, g0, g1, g2, g3, d0, d1, d2, d3,
                  r0, r1, r2, r3, sg0, sg1, sg2, sg3, ss0, ss1, ss2, ss3,
                  se0, se1, se2, se3, acc):
    cid = lax.axis_index("c")
    sid = lax.axis_index("s")
    wid = cid * NS + sid

    pltpu.sync_copy(zeros_hbm.at[pl.ds(sid * ROWS_PER_TILE, ROWS_PER_TILE)],
                    acc.at[pl.ds(sid * ROWS_PER_TILE, ROWS_PER_TILE)])
    plsc.subcore_barrier()

    def edata_start(c, ebuf, se):
        off = pl.multiple_of((wid * N_CHUNKS + c) * (3 * CHUNK), 8)
        pltpu.async_copy(edata_hbm.at[pl.ds(off, 3 * CHUNK)], ebuf, se)

    def edata_wait(c, ebuf, se):
        off = pl.multiple_of((wid * N_CHUNKS + c) * (3 * CHUNK), 8)
        pltpu.make_async_copy(edata_hbm.at[pl.ds(off, 3 * CHUNK)], ebuf,
                              se).wait()

    def gather_start(ebuf, gbuf, dbuf, rows, sg):
        for j in range(CHUNK // 16):
            gbuf[pl.ds(j * 16, 16)] = ebuf[pl.ds(j * 16, 16)]
            dbuf[pl.ds(j * 16, 16)] = ebuf[pl.ds(CHUNK + j * 16, 16)]
        pltpu.async_copy(table_hbm.at[gbuf], rows, sg)

    def stage_and_gather(c, ebuf, gbuf, dbuf, rows, sg, se):
        edata_start(c, ebuf, se)
        edata_wait(c, ebuf, se)
        gather_start(ebuf, gbuf, dbuf, rows, sg)

    def scale(ebuf, rows):
        def scale_body(g, _2):
            w16 = plsc.bitcast(ebuf[pl.ds(2 * CHUNK + g * 16, 16)],
                               jnp.float32)
            for j in range(16):
                ws = w16[j]
                k = g * 16 + j
                for col in range(H // 16):
                    rows[k, pl.ds(col * 16, 16)] = (
                        rows[k, pl.ds(col * 16, 16)] * ws)
            return _2

        lax.fori_loop(0, CHUNK // 16, scale_body, 0)

    def process(ebuf, gbuf, dbuf, rows, sg, ss):
        pltpu.make_async_copy(table_hbm.at[gbuf], rows, sg).wait()
        scale(ebuf, rows)
        pltpu.async_copy(rows, acc.at[dbuf], ss, add=True)

    def scatter_wait(dbuf, rows, ss):
        pltpu.make_async_copy(rows, acc.at[dbuf], ss).wait()

    bufs = [(e0, g0, d0, r0, sg0, ss0, se0), (e1, g1, d1, r1, sg1, ss1, se1),
            (e2, g2, d2, r2, sg2, ss2, se2), (e3, g3, d3, r3, sg3, ss3, se3)]
    DEPTH = 4
    n_full = N_CHUNKS // DEPTH

    for b in range(DEPTH):
        eb, gb, db, rb, sgb, _, seb = bufs[b]
        stage_and_gather(b, eb, gb, db, rb, sgb, seb)

    def rot_body(k, carry):
        for b in range(DEPTH):
            eb, gb, db, rb, sgb, ssb, seb = bufs[b]
            process(eb, gb, db, rb, sgb, ssb)
            if b == 0:
                edata_start(DEPTH * k + DEPTH, eb, seb)
            else:
                @pl.when(k < n_full - 1)
                def _estart(eb=eb, seb=seb, c=DEPTH * k + DEPTH + b):
                    edata_start(c, eb, seb)
        eb, gb, db, rb, sgb, ssb, seb = bufs[0]
        edata_wait(DEPTH * k + DEPTH, eb, seb)
        scatter_wait(db, rb, ssb)
        gather_start(eb, gb, db, rb, sgb)
        for b in range(1, DEPTH):
            eb, gb, db, rb, sgb, ssb, seb = bufs[b]

            @pl.when(k < n_full - 1)
            def _prefetch(eb=eb, gb=gb, db=db, rb=rb, sgb=sgb, ssb=ssb,
                          seb=seb, c=DEPTH * k + DEPTH + b):
                edata_wait(c, eb, seb)
                scatter_wait(db, rb, ssb)
                gather_start(eb, gb, db, rb, sgb)

        return carry

    lax.fori_loop(0, n_full, rot_body, 0)
    e_l, g_l, d_l, r_l, sg_l, ss_l, _ = bufs[0]
    process(e_l, g_l, d_l, r_l, sg_l, ss_l)
    for b in range(DEPTH):
        _, _, db, rb, _, ssb, _ = bufs[b]
        scatter_wait(db, rb, ssb)
    plsc.subcore_barrier()

    pltpu.sync_copy(acc.at[pl.ds(sid * ROWS_PER_TILE, ROWS_PER_TILE)],
                    out_hbm.at[cid, pl.ds(sid * ROWS_PER_TILE, ROWS_PER_TILE)])


BN = 1000


def _mm1_body(x_ref, w_ref, xr_ref, self_ref):
    y = jnp.dot(x_ref[...], w_ref[...], preferred_element_type=jnp.float32)
    for r in range(R):
        xr_ref[r] = y[:, r * H:(r + 1) * H]
    self_ref[...] = y[:, R * H:]


def _tc_transform(x, w_full):
    return pl.pallas_call(
        _mm1_body,
        grid=(N // BN,),
        in_specs=[
            pl.BlockSpec((BN, IN), lambda i: (i, 0)),
            pl.BlockSpec((IN, R * H + H), lambda i: (0, 0)),
        ],
        out_specs=[
            pl.BlockSpec((R, BN, H), lambda i: (0, i, 0)),
            pl.BlockSpec((BN, H), lambda i: (i, 0)),
        ],
        out_shape=[
            jax.ShapeDtypeStruct((R, N, H), jnp.float32),
            jax.ShapeDtypeStruct((N, H), jnp.float32),
        ],
    )(x, w_full)


def _mm2_body(self_ref, agg_ref, b_ref, w_ref, xr_ref, self2_ref):
    h = jnp.maximum(
        self_ref[...] + agg_ref[0] + agg_ref[1] + b_ref[...], 0.0)
    y = jnp.dot(h, w_ref[...], preferred_element_type=jnp.float32)
    for r in range(R):
        xr_ref[r] = y[:, r * H:(r + 1) * H]
    self2_ref[...] = y[:, R * H:]


def _tc_relu_transform(self1, agg, b, w_full):
    return pl.pallas_call(
        _mm2_body,
        grid=(N // BN,),
        in_specs=[
            pl.BlockSpec((BN, H), lambda i: (i, 0)),
            pl.BlockSpec((NC, BN, H), lambda i: (0, i, 0)),
            pl.BlockSpec((1, H), lambda i: (0, 0)),
            pl.BlockSpec((H, R * H + H), lambda i: (0, 0)),
        ],
        out_specs=[
            pl.BlockSpec((R, BN, H), lambda i: (0, i, 0)),
            pl.BlockSpec((BN, H), lambda i: (i, 0)),
        ],
        out_shape=[
            jax.ShapeDtypeStruct((R, N, H), jnp.float32),
            jax.ShapeDtypeStruct((N, H), jnp.float32),
        ],
    )(self1, agg, b, w_full)


def _final_body(self_ref, agg_ref, b_ref, batch_ref, h_ref, pool_ref):
    i = pl.program_id(0)
    h = jnp.maximum(
        self_ref[...] + agg_ref[0] + agg_ref[1] + b_ref[...], 0.0)
    h_ref[...] = h

    @pl.when(i == 0)
    def _():
        pool_ref[...] = jnp.full((G, H), -jnp.inf, jnp.float32)

    bids = batch_ref[...]
    for g in range(G):
        mg = jnp.max(jnp.where(bids == g, h, -jnp.inf), axis=0,
                     keepdims=True)
        pool_ref[pl.ds(g, 1), :] = jnp.maximum(pool_ref[pl.ds(g, 1), :], mg)


def _tc_final(self2, agg, b, batch2d):
    return pl.pallas_call(
        _final_body,
        grid=(N // BN,),
        in_specs=[
            pl.BlockSpec((BN, H), lambda i: (i, 0)),
            pl.BlockSpec((NC, BN, H), lambda i: (0, i, 0)),
            pl.BlockSpec((1, H), lambda i: (0, 0)),
            pl.BlockSpec((BN, 1), lambda i: (i, 0)),
        ],
        out_specs=[
            pl.BlockSpec((BN, H), lambda i: (i, 0)),
            pl.BlockSpec((G, H), lambda i: (0, 0)),
        ],
        out_shape=[
            jax.ShapeDtypeStruct((N, H), jnp.float32),
            jax.ShapeDtypeStruct((G, H), jnp.float32),
        ],
    )(self2, agg, b, batch2d)


def kernel(x, edge_index, edge_type, batch, W1, root1, b1, W2, root2, b2):
    src = edge_index[0]
    dst = edge_index[1]
    gidx = edge_type * N + src
    cidx = dst * R + edge_type

    zeros_nr = jnp.zeros((NR_PAD,), jnp.float32)
    zeros_nh = jnp.zeros((N_PAD, H), jnp.float32)

    w_edge = _sc_weights(cidx, zeros_nr)
    w_bits = lax.bitcast_convert_type(w_edge, jnp.int32)
    edata = (jnp.stack([gidx, dst, w_bits])
             .reshape(3, NW * N_CHUNKS, CHUNK)
             .transpose(1, 0, 2)
             .reshape(-1))

    wfull1 = jnp.concatenate(
        [W1.transpose(1, 0, 2).reshape(IN, R * H), root1], axis=1)
    xr1, self1 = _tc_transform(x, wfull1)
    agg1 = _sc_aggregate(xr1.reshape(NR, H), edata, zeros_nh)

    wfull2 = jnp.concatenate(
        [W2.transpose(1, 0, 2).reshape(H, R * H), root2], axis=1)
    xr2, self2 = _tc_relu_transform(self1, agg1, b1.reshape(1, H), wfull2)
    agg2 = _sc_aggregate(xr2.reshape(NR, H), edata, zeros_nh)

    h, pooled = _tc_final(self2, agg2, b2.reshape(1, H),
                          batch.reshape(N, 1))
    return (h, pooled)


# speedup vs baseline: 63.8255x; 1.0529x over previous
import functools

import jax
import jax.numpy as jnp
from jax import lax
from jax.experimental import pallas as pl
from jax.experimental.pallas import tpu as pltpu
from jax.experimental.pallas import tpu_sc as plsc

N = 10000
E = 320000
IN = 128
H = 128
R = 8
G = 16
NR = N * R
NR_PAD = 81920

_INFO = plsc.get_sparse_core_info()
NC = _INFO.num_cores
NS = _INFO.num_subcores
NW = NC * NS

E_PER_TILE = E // NW
E_PER_SUB = E // NS
CHUNK = 80
N_CHUNKS = E_PER_TILE // CHUNK
N_CHUNKS_CNT = E_PER_SUB // CHUNK
N_PAD = 10240
ROWS_PER_TILE = N_PAD // NS
ZNR_PER_TILE = NR_PAD // NS

_MESH = plsc.VectorSubcoreMesh(core_axis_name="c", subcore_axis_name="s")
_SC_PARAMS = pltpu.CompilerParams(needs_layout_passes=False)


@functools.partial(
    pl.kernel,
    out_type=jax.ShapeDtypeStruct((E,), jnp.float32),
    mesh=_MESH,
    scratch_types=[
        pltpu.VMEM((E_PER_SUB,), jnp.int32),
        pltpu.VMEM((CHUNK,), jnp.int32),
        pltpu.VMEM((CHUNK,), jnp.int32),
        pltpu.VMEM((CHUNK,), jnp.float32),
        pltpu.VMEM((NR_PAD,), jnp.float32),
        pltpu.VMEM((E_PER_TILE,), jnp.float32),
        pltpu.SemaphoreType.DMA,
        pltpu.SemaphoreType.DMA,
        pltpu.VMEM_SHARED((NR_PAD,), jnp.float32),
    ],
    compiler_params=_SC_PARAMS,
)
def _sc_weights(cidx_hbm, zeros_hbm, w_hbm, cbuf, ibuf_a, ibuf_b, ones,
                cpriv, wbuf, sa, sb, scnt):
    cid = lax.axis_index("c")
    sid = lax.axis_index("s")
    wid = cid * NS + sid

    pltpu.sync_copy(zeros_hbm.at[pl.ds(sid * ZNR_PER_TILE, ZNR_PER_TILE)],
                    scnt.at[pl.ds(sid * ZNR_PER_TILE, ZNR_PER_TILE)])
    for j in range(CHUNK // 16):
        ones[pl.ds(j * 16, 16)] = jnp.full((16,), 1.0, jnp.float32)
    pltpu.sync_copy(cidx_hbm.at[pl.ds(sid * E_PER_SUB, E_PER_SUB)], cbuf)
    plsc.subcore_barrier()

    def prep(c, ibuf):
        for j in range(CHUNK // 16):
            ibuf[pl.ds(j * 16, 16)] = cbuf[pl.ds(c * CHUNK + j * 16, 16)]

    def add_start(ibuf, sem):
        pltpu.async_copy(ones, scnt.at[ibuf], sem, add=True)

    def add_wait(ibuf, sem):
        pltpu.make_async_copy(ones, scnt.at[ibuf], sem).wait()

    prep(0, ibuf_a)
    add_start(ibuf_a, sa)
    prep(1, ibuf_b)
    add_start(ibuf_b, sb)

    def count_body(k, carry):
        add_wait(ibuf_a, sa)
        prep(2 * k + 2, ibuf_a)
        add_start(ibuf_a, sa)
        add_wait(ibuf_b, sb)
        prep(2 * k + 3, ibuf_b)
        add_start(ibuf_b, sb)
        return carry

    lax.fori_loop(0, N_CHUNKS_CNT // 2 - 1, count_body, 0)
    add_wait(ibuf_a, sa)
    add_wait(ibuf_b, sb)
    plsc.subcore_barrier()

    pltpu.sync_copy(scnt, cpriv)
    pltpu.sync_copy(cidx_hbm.at[pl.ds(wid * E_PER_TILE, E_PER_TILE)],
                    cbuf.at[pl.ds(0, E_PER_TILE)])

    def w_body(g, _):
        idx16 = cbuf[pl.ds(g * 16, 16)]
        c16 = plsc.load_gather(cpriv, [idx16])
        wbuf[pl.ds(g * 16, 16)] = 1.0 / jnp.maximum(c16, 1.0)
        return _

    lax.fori_loop(0, E_PER_TILE // 16, w_body, 0)
    pltpu.sync_copy(wbuf, w_hbm.at[pl.ds(wid * E_PER_TILE, E_PER_TILE)])


@functools.partial(
    pl.kernel,
    out_type=jax.ShapeDtypeStruct((NC, N_PAD, H), jnp.float32),
    mesh=_MESH,
    scratch_types=(
        [pltpu.VMEM((3 * CHUNK,), jnp.int32)] * 4 +
        [pltpu.VMEM((CHUNK,), jnp.int32)] * 4 +
        [pltpu.VMEM((CHUNK,), jnp.int32)] * 4 +
        [pltpu.VMEM((CHUNK, H), jnp.float32)] * 4 +
        [pltpu.SemaphoreType.DMA] * 4 +
        [pltpu.SemaphoreType.DMA] * 4 +
        [pltpu.SemaphoreType.DMA] * 4 +
        [pltpu.VMEM_SHARED((N_PAD, H), jnp.float32)]
    ),
    compiler_params=_SC_PARAMS,
)
def _sc_aggregate(table_hbm, edata_hbm, out_hbm,
                  e0, e1, e2, e3, g0, g1, g2, g3, d0, d1, d2, d3,
                  r0, r1, r2, r3, sg0, sg1, sg2, sg3, ss0, ss1, ss2, ss3,
                  se0, se1, se2, se3, acc):
    cid = lax.axis_index("c")
    sid = lax.axis_index("s")
    wid = cid * NS + sid

    zero16 = jnp.zeros((16,), jnp.float32)

    def zfill(q, carry):
        for col in range(H // 16):
            r0[q, pl.ds(col * 16, 16)] = zero16
        return carry

    lax.fori_loop(0, CHUNK, zfill, 0)
    for q in range(ROWS_PER_TILE // CHUNK):
        pltpu.sync_copy(
            r0, acc.at[pl.ds(sid * ROWS_PER_TILE + q * CHUNK, CHUNK)])
    plsc.subcore_barrier()

    def edata_start(c, ebuf, se):
        off = pl.multiple_of((wid * N_CHUNKS + c) * (3 * CHUNK), 8)
        pltpu.async_copy(edata_hbm.at[pl.ds(off, 3 * CHUNK)], ebuf, se)

    def edata_wait(c, ebuf, se):
        off = pl.multiple_of((wid * N_CHUNKS + c) * (3 * CHUNK), 8)
        pltpu.make_async_copy(edata_hbm.at[pl.ds(off, 3 * CHUNK)], ebuf,
                              se).wait()

    def gather_start(ebuf, gbuf, dbuf, rows, sg):
        for j in range(CHUNK // 16):
            gbuf[pl.ds(j * 16, 16)] = ebuf[pl.ds(j * 16, 16)]
            dbuf[pl.ds(j * 16, 16)] = ebuf[pl.ds(CHUNK + j * 16, 16)]
        pltpu.async_copy(table_hbm.at[gbuf], rows, sg)

    def stage_and_gather(c, ebuf, gbuf, dbuf, rows, sg, se):
        edata_start(c, ebuf, se)
        edata_wait(c, ebuf, se)
        gather_start(ebuf, gbuf, dbuf, rows, sg)

    def scale(ebuf, rows):
        def scale_body(g, _2):
            w16 = plsc.bitcast(ebuf[pl.ds(2 * CHUNK + g * 16, 16)],
                               jnp.float32)
            for j in range(16):
                ws = w16[j]
                k = g * 16 + j
                for col in range(H // 16):
                    rows[k, pl.ds(col * 16, 16)] = (
                        rows[k, pl.ds(col * 16, 16)] * ws)
            return _2

        lax.fori_loop(0, CHUNK // 16, scale_body, 0)

    def process(ebuf, gbuf, dbuf, rows, sg, ss):
        pltpu.make_async_copy(table_hbm.at[gbuf], rows, sg).wait()
        scale(ebuf, rows)
        pltpu.async_copy(rows, acc.at[dbuf], ss, add=True)

    def scatter_wait(dbuf, rows, ss):
        pltpu.make_async_copy(rows, acc.at[dbuf], ss).wait()

    bufs = [(e0, g0, d0, r0, sg0, ss0, se0), (e1, g1, d1, r1, sg1, ss1, se1),
            (e2, g2, d2, r2, sg2, ss2, se2), (e3, g3, d3, r3, sg3, ss3, se3)]
    DEPTH = 4
    n_full = N_CHUNKS // DEPTH

    for b in range(DEPTH):
        eb, gb, db, rb, sgb, _, seb = bufs[b]
        stage_and_gather(b, eb, gb, db, rb, sgb, seb)

    def rot_body(k, carry):
        for b in range(DEPTH):
            eb, gb, db, rb, sgb, ssb, seb = bufs[b]
            process(eb, gb, db, rb, sgb, ssb)
            if b == 0:
                edata_start(DEPTH * k + DEPTH, eb, seb)
            else:
                @pl.when(k < n_full - 1)
                def _estart(eb=eb, seb=seb, c=DEPTH * k + DEPTH + b):
                    edata_start(c, eb, seb)
        eb, gb, db, rb, sgb, ssb, seb = bufs[0]
        edata_wait(DEPTH * k + DEPTH, eb, seb)
        scatter_wait(db, rb, ssb)
        gather_start(eb, gb, db, rb, sgb)
        for b in range(1, DEPTH):
            eb, gb, db, rb, sgb, ssb, seb = bufs[b]

            @pl.when(k < n_full - 1)
            def _prefetch(eb=eb, gb=gb, db=db, rb=rb, sgb=sgb, ssb=ssb,
                          seb=seb, c=DEPTH * k + DEPTH + b):
                edata_wait(c, eb, seb)
                scatter_wait(db, rb, ssb)
                gather_start(eb, gb, db, rb, sgb)

        return carry

    lax.fori_loop(0, n_full, rot_body, 0)
    e_l, g_l, d_l, r_l, sg_l, ss_l, _ = bufs[0]
    process(e_l, g_l, d_l, r_l, sg_l, ss_l)
    for b in range(DEPTH):
        _, _, db, rb, _, ssb, _ = bufs[b]
        scatter_wait(db, rb, ssb)
    plsc.subcore_barrier()

    pltpu.sync_copy(acc.at[pl.ds(sid * ROWS_PER_TILE, ROWS_PER_TILE)],
                    out_hbm.at[cid, pl.ds(sid * ROWS_PER_TILE, ROWS_PER_TILE)])


BN = 2000


def _mm1_body(x_ref, w_ref, xr_ref, self_ref):
    y = jnp.dot(x_ref[...], w_ref[...], preferred_element_type=jnp.float32)
    for r in range(R):
        xr_ref[r] = y[:, r * H:(r + 1) * H]
    self_ref[...] = y[:, R * H:]


def _tc_transform(x, w_full):
    return pl.pallas_call(
        _mm1_body,
        grid=(N // BN,),
        in_specs=[
            pl.BlockSpec((BN, IN), lambda i: (i, 0)),
            pl.BlockSpec((IN, R * H + H), lambda i: (0, 0)),
        ],
        out_specs=[
            pl.BlockSpec((R, BN, H), lambda i: (0, i, 0)),
            pl.BlockSpec((BN, H), lambda i: (i, 0)),
        ],
        out_shape=[
            jax.ShapeDtypeStruct((R, N, H), jnp.float32),
            jax.ShapeDtypeStruct((N, H), jnp.float32),
        ],
    )(x, w_full)


def _mm2_body(self_ref, agg_ref, b_ref, w_ref, xr_ref, self2_ref):
    h = jnp.maximum(
        self_ref[...] + agg_ref[0] + agg_ref[1] + b_ref[...], 0.0)
    y = jnp.dot(h, w_ref[...], preferred_element_type=jnp.float32)
    for r in range(R):
        xr_ref[r] = y[:, r * H:(r + 1) * H]
    self2_ref[...] = y[:, R * H:]


def _tc_relu_transform(self1, agg, b, w_full):
    return pl.pallas_call(
        _mm2_body,
        grid=(N // BN,),
        in_specs=[
            pl.BlockSpec((BN, H), lambda i: (i, 0)),
            pl.BlockSpec((NC, BN, H), lambda i: (0, i, 0)),
            pl.BlockSpec((1, H), lambda i: (0, 0)),
            pl.BlockSpec((H, R * H + H), lambda i: (0, 0)),
        ],
        out_specs=[
            pl.BlockSpec((R, BN, H), lambda i: (0, i, 0)),
            pl.BlockSpec((BN, H), lambda i: (i, 0)),
        ],
        out_shape=[
            jax.ShapeDtypeStruct((R, N, H), jnp.float32),
            jax.ShapeDtypeStruct((N, H), jnp.float32),
        ],
    )(self1, agg, b, w_full)


def _final_body(self_ref, agg_ref, b_ref, batch_ref, h_ref, pool_ref):
    i = pl.program_id(0)
    h = jnp.maximum(
        self_ref[...] + agg_ref[0] + agg_ref[1] + b_ref[...], 0.0)
    h_ref[...] = h

    @pl.when(i == 0)
    def _():
        pool_ref[...] = jnp.full((G, H), -jnp.inf, jnp.float32)

    bids = batch_ref[...]
    for g in range(G):
        mg = jnp.max(jnp.where(bids == g, h, -jnp.inf), axis=0,
                     keepdims=True)
        pool_ref[pl.ds(g, 1), :] = jnp.maximum(pool_ref[pl.ds(g, 1), :], mg)


def _tc_final(self2, agg, b, batch2d):
    return pl.pallas_call(
        _final_body,
        grid=(N // BN,),
        in_specs=[
            pl.BlockSpec((BN, H), lambda i: (i, 0)),
            pl.BlockSpec((NC, BN, H), lambda i: (0, i, 0)),
            pl.BlockSpec((1, H), lambda i: (0, 0)),
            pl.BlockSpec((BN, 1), lambda i: (i, 0)),
        ],
        out_specs=[
            pl.BlockSpec((BN, H), lambda i: (i, 0)),
            pl.BlockSpec((G, H), lambda i: (0, 0)),
        ],
        out_shape=[
            jax.ShapeDtypeStruct((N, H), jnp.float32),
            jax.ShapeDtypeStruct((G, H), jnp.float32),
        ],
    )(self2, agg, b, batch2d)


def kernel(x, edge_index, edge_type, batch, W1, root1, b1, W2, root2, b2):
    src = edge_index[0]
    dst = edge_index[1]
    gidx = edge_type * N + src
    cidx = dst * R + edge_type

    zeros_nr = jnp.zeros((NR_PAD,), jnp.float32)

    w_edge = _sc_weights(cidx, zeros_nr)
    w_bits = lax.bitcast_convert_type(w_edge, jnp.int32)
    edata = (jnp.stack([gidx, dst, w_bits])
             .reshape(3, NW * N_CHUNKS, CHUNK)
             .transpose(1, 0, 2)
             .reshape(-1))

    wfull1 = jnp.concatenate(
        [W1.transpose(1, 0, 2).reshape(IN, R * H), root1], axis=1)
    xr1, self1 = _tc_transform(x, wfull1)
    agg1 = _sc_aggregate(xr1.reshape(NR, H), edata)

    wfull2 = jnp.concatenate(
        [W2.transpose(1, 0, 2).reshape(H, R * H), root2], axis=1)
    xr2, self2 = _tc_relu_transform(self1, agg1, b1.reshape(1, H), wfull2)
    agg2 = _sc_aggregate(xr2.reshape(NR, H), edata)

    h, pooled = _tc_final(self2, agg2, b2.reshape(1, H),
                          batch.reshape(N, 1))
    return (h, pooled)
